# Initial kernel scaffold; baseline (speedup 1.0000x reference)
import functools

import jax
import jax.numpy as jnp
from jax import lax
from jax.experimental import pallas as pl
from jax.experimental.pallas import tpu as pltpu
from jax.experimental.pallas import tpu_sc as plsc

F32 = jnp.float32

NC, NS = 2, 16
NW = NC * NS
DN = 16


def _sc_mesh():
    return plsc.VectorSubcoreMesh(core_axis_name="c", subcore_axis_name="s",
                                  num_cores=NC, num_subcores=NS)


def _gather_rows(table, idx, chunk=2000):
    M = idx.shape[0]
    b_w = M // NW
    n_chunks = b_w // chunk

    @functools.partial(
        pl.kernel,
        out_type=jax.ShapeDtypeStruct((M, DN), F32),
        mesh=_sc_mesh(),
        scratch_types=[pltpu.VMEM((chunk,), jnp.int32),
                       pltpu.VMEM((chunk, DN), F32),
                       pltpu.SemaphoreType.DMA],
    )
    def k(table_hbm, idx_hbm, out_hbm, idx_v, rows_v, sem):
        wid = lax.axis_index("c") * NS + lax.axis_index("s")
        base = wid * b_w

        def body(i, carry):
            off = base + i * chunk
            pltpu.sync_copy(idx_hbm.at[pl.ds(off, chunk)], idx_v)
            pltpu.async_copy(table_hbm.at[idx_v], rows_v, sem).wait()
            pltpu.sync_copy(rows_v, out_hbm.at[pl.ds(off, chunk)])
            return carry

        lax.fori_loop(0, n_chunks, body, 0)

    return k(table, idx)


def _segment_sum(e, dst, n_pad, chunk=5000):
    M = e.shape[0]
    b_w = M // NW
    n_chunks = b_w // chunk
    rpt = n_pad // NS
    zeros = jnp.zeros((rpt, DN), F32)

    @functools.partial(
        pl.kernel,
        out_type=jax.ShapeDtypeStruct((NC * n_pad, DN), F32),
        mesh=_sc_mesh(),
        scratch_types=[pltpu.VMEM((chunk,), jnp.int32),
                       pltpu.VMEM((chunk, DN), F32),
                       pltpu.VMEM_SHARED((n_pad, DN), F32),
                       pltpu.SemaphoreType.DMA],
    )
    def k(e_hbm, dst_hbm, z_hbm, out_hbm, idx_v, rows_v, acc_sh, sem):
        c = lax.axis_index("c")
        s = lax.axis_index("s")
        pltpu.sync_copy(z_hbm, acc_sh.at[pl.ds(s * rpt, rpt)])
        plsc.subcore_barrier()
        base = (c * NS + s) * b_w

        def body(i, carry):
            off = base + i * chunk
            pltpu.sync_copy(dst_hbm.at[pl.ds(off, chunk)], idx_v)
            pltpu.sync_copy(e_hbm.at[pl.ds(off, chunk)], rows_v)
            pltpu.sync_copy(rows_v, acc_sh.at[idx_v], add=True)
            return carry

        lax.fori_loop(0, n_chunks, body, 0)
        plsc.subcore_barrier()
        pltpu.sync_copy(acc_sh.at[pl.ds(s * rpt, rpt)],
                        out_hbm.at[pl.ds(c * n_pad + s * rpt, rpt)])

    return k(e, dst, zeros)


def _tc_mlp(n_out, br, pieces, b1, hidden, out_wt, out_b, final=None):
    grid = n_out // br
    in_specs = []
    operands = []
    for arr, blk_off, _ in pieces:
        d = arr.shape[1]
        in_specs.append(pl.BlockSpec((br, d), lambda i, o=blk_off: (i + o, 0)))
        operands.append(arr)
    for arr, _, wp in pieces:
        in_specs.append(pl.BlockSpec(wp.shape, lambda i: (0, 0)))
        operands.append(wp)

    def whole(a):
        in_specs.append(pl.BlockSpec(a.shape, lambda i: (0, 0)))
        operands.append(a)

    whole(b1)
    for w, b in hidden:
        whole(w)
        whole(b)
    whole(out_wt)
    whole(out_b)

    n_p = len(pieces)
    n_h = len(hidden)
    out_width = out_wt.shape[1]

    def body(*refs):
        xs = refs[:n_p]
        wps = refs[n_p:2 * n_p]
        k = 2 * n_p
        b1r = refs[k]
        k += 1
        hs = []
        for _ in range(n_h):
            hs.append((refs[k], refs[k + 1]))
            k += 2
        owr, obr = refs[k], refs[k + 1]
        out_ref = refs[k + 2]
        acc = None
        for xr, wr in zip(xs, wps):
            xv = xr[...]
            wv = wr[...]
            if xv.shape[1] == 1:
                t = xv * wv
            else:
                t = jnp.dot(xv, wv, preferred_element_type=F32)
            acc = t if acc is None else acc + t
        h = jnp.maximum(acc + b1r[...], 0.0)
        for wr2, br2 in hs:
            h = jnp.maximum(
                jnp.dot(h, wr2[...], preferred_element_type=F32) + br2[...], 0.0)
        o = jnp.dot(h, owr[...], preferred_element_type=F32) + obr[...]
        if final == "sigmoid":
            o = jax.nn.sigmoid(o)
        out_ref[...] = o

    return pl.pallas_call(
        body,
        grid=(grid,),
        in_specs=in_specs,
        out_specs=pl.BlockSpec((br, out_width), lambda i: (i, 0)),
        out_shape=jax.ShapeDtypeStruct((n_out, out_width), F32),
    )(*operands)


def _pad_rows(wt, rows):
    return jnp.pad(wt, ((0, rows - wt.shape[0]), (0, 0)))


def _pad_cols(wt, cols):
    return jnp.pad(wt, ((0, 0), (0, cols - wt.shape[1])))


def kernel(x, edge_index, edge_attr, params):
    n_nodes = x.shape[0]
    n_edges = edge_index.shape[1]
    n_pad = ((n_nodes + NS * 8 - 1) // (NS * 8)) * (NS * 8)

    dst = edge_index[1].astype(jnp.int32)
    src = edge_index[0].astype(jnp.int32)
    idx_cat = jnp.concatenate([dst, src])

    xp = jnp.zeros((n_pad, DN), F32).at[:n_nodes, :3].set(x)
    ea = edge_attr.astype(F32)

    BR_E = 8000
    BR_N = n_pad // 16

    def rel_call(p, e_pieces, gtab):
        (W1, b1), (W2, b2), (W3, b3), (W4, b4) = p
        W1t = W1.T
        pieces = [(gtab, 0, _pad_rows(W1t[0:3], DN)),
                  (gtab, n_edges // BR_E, _pad_rows(W1t[3:6], DN))]
        col = 6
        for arr, d in e_pieces:
            wp = W1t[col:col + d]
            col += d
            if arr.shape[1] != d:
                wp = _pad_rows(wp, arr.shape[1])
            pieces.append((arr, 0, wp))
        hidden = [(W2.T, b2[None, :]), (W3.T, b3[None, :])]
        return _tc_mlp(n_edges, BR_E, pieces, b1[None, :], hidden,
                       _pad_cols(W4.T, DN), _pad_cols(b4[None, :], DN))

    def obj_call(p, xtab, agg):
        (W1, b1), (W2, b2), (W3, b3), (W4, b4) = p
        W1t = W1.T
        eo = W1t.shape[0] - 3
        wx = _pad_rows(W1t[0:3], DN)
        wa = _pad_rows(W1t[3:3 + eo], DN)
        pieces = [(xtab, 0, wx), (agg, 0, wa), (agg, n_pad // BR_N, wa)]
        hidden = [(W2.T, b2[None, :]), (W3.T, b3[None, :])]
        return _tc_mlp(n_pad, BR_N, pieces, b1[None, :], hidden,
                       _pad_cols(W4.T, DN), _pad_cols(b4[None, :], DN))

    def head_call(p, pieces_in, n_out, br, final):
        (W1, b1), (W2, b2), (W3, b3), (W4, b4) = p
        W1t = W1.T
        pieces = []
        col = 0
        for arr, d in pieces_in:
            wp = W1t[col:col + d]
            col += d
            if arr.shape[1] != d:
                wp = _pad_rows(wp, arr.shape[1])
            pieces.append((arr, 0, wp))
        hidden = [(W2.T, b2[None, :]), (W3.T, b3[None, :])]
        return _tc_mlp(n_out, br, pieces, b1[None, :], hidden,
                       W4.T, b4[None, :], final=final)

    g0 = _gather_rows(xp, idx_cat)
    e1 = rel_call(params['in_w1']['rel'], [(ea, 4)], g0)
    a1 = _segment_sum(e1, dst, n_pad)
    x1 = obj_call(params['in_w1']['obj'], xp, a1)

    g1 = _gather_rows(x1, idx_cat)
    e2 = rel_call(params['in_w2']['rel'], [(e1, 4)], g1)

    ew = head_call(params['W'], [(ea, 4), (e1, 4), (e2, 4)],
                   n_edges, BR_E, "sigmoid")

    ec1 = rel_call(params['in_c1']['rel'],
                   [(ew, 1), (ea, 4), (e1, 4), (e2, 4)], g0)
    ac1 = _segment_sum(ec1, dst, n_pad)
    xc1 = obj_call(params['in_c1']['obj'], xp, ac1)

    gc2 = _gather_rows(xc1, idx_cat)
    ec2 = rel_call(params['in_c2']['rel'], [(ec1, 8)], gc2)
    ac2 = _segment_sum(ec2, dst, n_pad)
    xc2 = obj_call(params['in_c2']['obj'], xc1, ac2)

    gc3 = _gather_rows(xc2, idx_cat)
    ec3 = rel_call(params['in_c3']['rel'], [(ec2, 8)], gc3)
    ac3 = _segment_sum(ec3, dst, n_pad)
    xc3 = obj_call(params['in_c3']['obj'], xc2, ac3)

    node_pieces = [(xp, 3), (xc1, 3), (xc2, 3), (xc3, 3)]
    beta = head_call(params['B'], node_pieces, n_pad, BR_N, "sigmoid")
    xc = head_call(params['X'], node_pieces, n_pad, BR_N, None)

    return ew, xc[:n_nodes], beta[:n_nodes]


# speedup vs baseline: 2.3148x; 2.3148x over previous
import functools

import jax
import jax.numpy as jnp
from jax import lax
from jax.experimental import pallas as pl
from jax.experimental.pallas import tpu as pltpu
from jax.experimental.pallas import tpu_sc as plsc

F32 = jnp.float32

NC, NS = 2, 16
NW = NC * NS
DN = 16
DE = 8


def _sc_mesh():
    return plsc.VectorSubcoreMesh(core_axis_name="c", subcore_axis_name="s",
                                  num_cores=NC, num_subcores=NS)


def _gather_rows(table, idx, chunk=2000):
    M = idx.shape[0]
    b_w = M // NW
    n_chunks = b_w // chunk

    @functools.partial(
        pl.kernel,
        out_type=jax.ShapeDtypeStruct((M, DN), F32),
        mesh=_sc_mesh(),
        scratch_types=[pltpu.VMEM((chunk,), jnp.int32),
                       pltpu.VMEM((chunk, DN), F32),
                       pltpu.SemaphoreType.DMA],
        compiler_params=pltpu.CompilerParams(use_tc_tiling_on_sc=False),
    )
    def k(table_hbm, idx_hbm, out_hbm, idx_v, rows_v, sem):
        wid = lax.axis_index("c") * NS + lax.axis_index("s")
        base = wid * b_w

        def body(i, carry):
            off = base + i * chunk
            pltpu.sync_copy(idx_hbm.at[pl.ds(off, chunk)], idx_v)
            pltpu.async_copy(table_hbm.at[idx_v], rows_v, sem).wait()
            pltpu.sync_copy(rows_v, out_hbm.at[pl.ds(off, chunk)])
            return carry

        lax.fori_loop(0, n_chunks, body, 0)

    return k(table, idx)


def _segment_sum(e, dst, n_pad, chunk=5000):
    M = e.shape[0]
    b_w = M // NW
    n_chunks = b_w // chunk
    rpt = n_pad // NS
    zeros = jnp.zeros((rpt, DE), F32)

    @functools.partial(
        pl.kernel,
        out_type=jax.ShapeDtypeStruct((NC * n_pad, DE), F32),
        mesh=_sc_mesh(),
        scratch_types=[pltpu.VMEM((chunk,), jnp.int32),
                       pltpu.VMEM((chunk, DE), F32),
                       pltpu.VMEM_SHARED((n_pad, DE), F32),
                       pltpu.SemaphoreType.DMA],
        compiler_params=pltpu.CompilerParams(use_tc_tiling_on_sc=False),
    )
    def k(e_hbm, dst_hbm, z_hbm, out_hbm, idx_v, rows_v, acc_sh, sem):
        c = lax.axis_index("c")
        s = lax.axis_index("s")
        pltpu.sync_copy(z_hbm, acc_sh.at[pl.ds(s * rpt, rpt)])
        plsc.subcore_barrier()
        base = (c * NS + s) * b_w

        def body(i, carry):
            off = base + i * chunk
            pltpu.sync_copy(dst_hbm.at[pl.ds(off, chunk)], idx_v)
            pltpu.sync_copy(e_hbm.at[pl.ds(off, chunk)], rows_v)
            pltpu.sync_copy(rows_v, acc_sh.at[idx_v], add=True)
            return carry

        lax.fori_loop(0, n_chunks, body, 0)
        plsc.subcore_barrier()
        pltpu.sync_copy(acc_sh.at[pl.ds(s * rpt, rpt)],
                        out_hbm.at[pl.ds(c * n_pad + s * rpt, rpt)])

    return k(e, dst, zeros)


def _tc_mlp(n_out, br, pieces, b1, hidden, out_wt, out_b, final=None):
    grid = n_out // br
    in_specs = []
    operands = []
    for arr, blk_off, _ in pieces:
        d = arr.shape[1]
        in_specs.append(pl.BlockSpec((br, d), lambda i, o=blk_off: (i + o, 0)))
        operands.append(arr)
    for arr, _, wp in pieces:
        in_specs.append(pl.BlockSpec(wp.shape, lambda i: (0, 0)))
        operands.append(wp)

    def whole(a):
        in_specs.append(pl.BlockSpec(a.shape, lambda i: (0, 0)))
        operands.append(a)

    whole(b1)
    for w, b in hidden:
        whole(w)
        whole(b)
    whole(out_wt)
    whole(out_b)

    n_p = len(pieces)
    n_h = len(hidden)
    out_width = out_wt.shape[1]

    def body(*refs):
        xs = refs[:n_p]
        wps = refs[n_p:2 * n_p]
        k = 2 * n_p
        b1r = refs[k]
        k += 1
        hs = []
        for _ in range(n_h):
            hs.append((refs[k], refs[k + 1]))
            k += 2
        owr, obr = refs[k], refs[k + 1]
        out_ref = refs[k + 2]
        acc = None
        for xr, wr in zip(xs, wps):
            xv = xr[...]
            wv = wr[...]
            if xv.shape[1] == 1:
                t = xv * wv
            else:
                t = jnp.dot(xv, wv, preferred_element_type=F32)
            acc = t if acc is None else acc + t
        h = jnp.maximum(acc + b1r[...], 0.0)
        for wr2, br2 in hs:
            h = jnp.maximum(
                jnp.dot(h, wr2[...], preferred_element_type=F32) + br2[...], 0.0)
        o = jnp.dot(h, owr[...], preferred_element_type=F32) + obr[...]
        if final == "sigmoid":
            o = jax.nn.sigmoid(o)
        out_ref[...] = o

    return pl.pallas_call(
        body,
        grid=(grid,),
        in_specs=in_specs,
        out_specs=pl.BlockSpec((br, out_width), lambda i: (i, 0)),
        out_shape=jax.ShapeDtypeStruct((n_out, out_width), F32),
    )(*operands)


def _pad_rows(wt, rows):
    return jnp.pad(wt, ((0, rows - wt.shape[0]), (0, 0)))


def _pad_cols(wt, cols):
    return jnp.pad(wt, ((0, 0), (0, cols - wt.shape[1])))


def kernel(x, edge_index, edge_attr, params):
    n_nodes = x.shape[0]
    n_edges = edge_index.shape[1]
    n_pad = ((n_nodes + NS * 8 - 1) // (NS * 8)) * (NS * 8)

    dst = edge_index[1].astype(jnp.int32)
    src = edge_index[0].astype(jnp.int32)
    idx_cat = jnp.concatenate([dst, src])

    xp = jnp.zeros((n_pad, DN), F32).at[:n_nodes, :3].set(x)
    ea = edge_attr.astype(F32)

    BR_E = 2000
    BR_N = n_pad // 16

    def rel_call(p, e_pieces, gtab):
        (W1, b1), (W2, b2), (W3, b3), (W4, b4) = p
        W1t = W1.T
        pieces = [(gtab, 0, _pad_rows(W1t[0:3], DN)),
                  (gtab, n_edges // BR_E, _pad_rows(W1t[3:6], DN))]
        col = 6
        for arr, d in e_pieces:
            wp = W1t[col:col + d]
            col += d
            if arr.shape[1] != d:
                wp = _pad_rows(wp, arr.shape[1])
            pieces.append((arr, 0, wp))
        hidden = [(W2.T, b2[None, :]), (W3.T, b3[None, :])]
        return _tc_mlp(n_edges, BR_E, pieces, b1[None, :], hidden,
                       _pad_cols(W4.T, DE), _pad_cols(b4[None, :], DE))

    def obj_call(p, xtab, agg):
        (W1, b1), (W2, b2), (W3, b3), (W4, b4) = p
        W1t = W1.T
        eo = W1t.shape[0] - 3
        wx = _pad_rows(W1t[0:3], DN)
        wa = _pad_rows(W1t[3:3 + eo], DE)
        pieces = [(xtab, 0, wx), (agg, 0, wa), (agg, n_pad // BR_N, wa)]
        hidden = [(W2.T, b2[None, :]), (W3.T, b3[None, :])]
        return _tc_mlp(n_pad, BR_N, pieces, b1[None, :], hidden,
                       _pad_cols(W4.T, DN), _pad_cols(b4[None, :], DN))

    def head_call(p, pieces_in, n_out, br, final):
        (W1, b1), (W2, b2), (W3, b3), (W4, b4) = p
        W1t = W1.T
        pieces = []
        col = 0
        for arr, d in pieces_in:
            wp = W1t[col:col + d]
            col += d
            if arr.shape[1] != d:
                wp = _pad_rows(wp, arr.shape[1])
            pieces.append((arr, 0, wp))
        hidden = [(W2.T, b2[None, :]), (W3.T, b3[None, :])]
        return _tc_mlp(n_out, br, pieces, b1[None, :], hidden,
                       W4.T, b4[None, :], final=final)

    g0 = _gather_rows(xp, idx_cat)
    e1 = rel_call(params['in_w1']['rel'], [(ea, 4)], g0)
    a1 = _segment_sum(e1, dst, n_pad)
    x1 = obj_call(params['in_w1']['obj'], xp, a1)

    g1 = _gather_rows(x1, idx_cat)
    e2 = rel_call(params['in_w2']['rel'], [(e1, 4)], g1)

    ew = head_call(params['W'], [(ea, 4), (e1, 4), (e2, 4)],
                   n_edges, BR_E, "sigmoid")

    ec1 = rel_call(params['in_c1']['rel'],
                   [(ew, 1), (ea, 4), (e1, 4), (e2, 4)], g0)
    ac1 = _segment_sum(ec1, dst, n_pad)
    xc1 = obj_call(params['in_c1']['obj'], xp, ac1)

    gc2 = _gather_rows(xc1, idx_cat)
    ec2 = rel_call(params['in_c2']['rel'], [(ec1, 8)], gc2)
    ac2 = _segment_sum(ec2, dst, n_pad)
    xc2 = obj_call(params['in_c2']['obj'], xc1, ac2)

    gc3 = _gather_rows(xc2, idx_cat)
    ec3 = rel_call(params['in_c3']['rel'], [(ec2, 8)], gc3)
    ac3 = _segment_sum(ec3, dst, n_pad)
    xc3 = obj_call(params['in_c3']['obj'], xc2, ac3)

    node_pieces = [(xp, 3), (xc1, 3), (xc2, 3), (xc3, 3)]
    beta = head_call(params['B'], node_pieces, n_pad, BR_N, "sigmoid")
    xc = head_call(params['X'], node_pieces, n_pad, BR_N, None)

    return ew, xc[:n_nodes], beta[:n_nodes]


# speedup vs baseline: 3.8621x; 1.6684x over previous
import functools

import jax
import jax.numpy as jnp
from jax import lax
from jax.experimental import pallas as pl
from jax.experimental.pallas import tpu as pltpu
from jax.experimental.pallas import tpu_sc as plsc

F32 = jnp.float32
BF16 = jnp.bfloat16

NC, NS = 2, 16
NW = NC * NS
DN = 16


def _sc_mesh():
    return plsc.VectorSubcoreMesh(core_axis_name="c", subcore_axis_name="s",
                                  num_cores=NC, num_subcores=NS)


def _gather_rows(table, idx, chunk=2000):
    M = idx.shape[0]
    b_w = M // NW
    n_chunks = b_w // chunk

    @functools.partial(
        pl.kernel,
        out_type=jax.ShapeDtypeStruct((M, DN), F32),
        mesh=_sc_mesh(),
        scratch_types=[pltpu.VMEM((chunk,), jnp.int32),
                       pltpu.VMEM((chunk, DN), F32),
                       pltpu.SemaphoreType.DMA],
        compiler_params=pltpu.CompilerParams(use_tc_tiling_on_sc=False),
    )
    def k(table_hbm, idx_hbm, out_hbm, idx_v, rows_v, sem):
        wid = lax.axis_index("c") * NS + lax.axis_index("s")
        base = wid * b_w

        def body(i, carry):
            off = base + i * chunk
            pltpu.sync_copy(idx_hbm.at[pl.ds(off, chunk)], idx_v)
            pltpu.async_copy(table_hbm.at[idx_v], rows_v, sem).wait()
            pltpu.sync_copy(rows_v, out_hbm.at[pl.ds(off, chunk)])
            return carry

        lax.fori_loop(0, n_chunks, body, 0)

    return k(table, idx)


def _segment_sum(e, dst, n_pad, chunk=1000):
    M = e.shape[0]
    b_w = M // NW
    n_chunks = b_w // chunk
    rpt = n_pad // NS
    zeros = jnp.zeros((rpt, DN), F32)

    @functools.partial(
        pl.kernel,
        out_type=jax.ShapeDtypeStruct((NC * n_pad, DN), F32),
        mesh=_sc_mesh(),
        scratch_types=[pltpu.VMEM((chunk,), jnp.int32),
                       pltpu.VMEM((chunk, DN), F32),
                       pltpu.VMEM_SHARED((n_pad, DN), F32),
                       pltpu.SemaphoreType.DMA],
        compiler_params=pltpu.CompilerParams(use_tc_tiling_on_sc=False),
    )
    def k(e_hbm, dst_hbm, z_hbm, out_hbm, idx_v, rows_v, acc_sh, sem):
        c = lax.axis_index("c")
        s = lax.axis_index("s")
        pltpu.sync_copy(z_hbm, acc_sh.at[pl.ds(s * rpt, rpt)])
        plsc.subcore_barrier()
        base = (c * NS + s) * b_w

        def body(i, carry):
            off = base + i * chunk
            pltpu.sync_copy(dst_hbm.at[pl.ds(off, chunk)], idx_v)
            pltpu.sync_copy(e_hbm.at[pl.ds(off, chunk)], rows_v)
            pltpu.sync_copy(rows_v, acc_sh.at[idx_v], add=True)
            return carry

        lax.fori_loop(0, n_chunks, body, 0)
        plsc.subcore_barrier()
        pltpu.sync_copy(acc_sh.at[pl.ds(s * rpt, rpt)],
                        out_hbm.at[pl.ds(c * n_pad + s * rpt, rpt)])

    return k(e, dst, zeros)


def _tc_mlp(n_out, br, pieces, b1, hidden, out_wt, out_b, final=None,
            out_pack=1, narrow_width=0):
    grid = n_out // br
    in_specs = []
    operands = []
    for arr, blk_off, _, pack in pieces:
        if pack > 1:
            in_specs.append(
                pl.BlockSpec((br // pack, 128),
                             lambda i, o=blk_off: (i + o, 0)))
        else:
            in_specs.append(
                pl.BlockSpec((br, arr.shape[1]),
                             lambda i, o=blk_off: (i + o, 0)))
        operands.append(arr)
    for arr, _, wp, _ in pieces:
        in_specs.append(pl.BlockSpec(wp.shape, lambda i: (0, 0)))
        operands.append(wp)

    def whole(a):
        in_specs.append(pl.BlockSpec(a.shape, lambda i: (0, 0)))
        operands.append(a)

    whole(b1)
    for w, b in hidden:
        whole(w)
        whole(b)
    whole(out_wt)
    whole(out_b)

    n_p = len(pieces)
    n_h = len(hidden)
    out_width = out_wt.shape[1]
    packs = [p[3] for p in pieces]

    packed = any(p > 1 for p in packs)

    def mlp_chain(xvs, wvs, b1v, hvs, owv, obv):
        acc = None
        for xv, wv in zip(xvs, wvs):
            t = jnp.dot(xv.astype(BF16), wv, preferred_element_type=F32)
            acc = t if acc is None else acc + t
        h = jnp.maximum(acc + b1v, 0.0)
        for wv2, bv2 in hvs:
            h = jnp.maximum(
                jnp.dot(h.astype(BF16), wv2,
                        preferred_element_type=F32) + bv2, 0.0)
        o = jnp.dot(h.astype(BF16), owv, preferred_element_type=F32) + obv
        if final == "sigmoid":
            o = jax.nn.sigmoid(o)
        return o

    def body(*refs):
        xs = refs[:n_p]
        wps = refs[n_p:2 * n_p]
        k = 2 * n_p
        b1r = refs[k]
        k += 1
        hs = []
        for _ in range(n_h):
            hs.append((refs[k], refs[k + 1]))
            k += 2
        owr, obr = refs[k], refs[k + 1]
        out_refs = refs[k + 2:]
        wvs = [wr[...] for wr in wps]
        b1v = b1r[...]
        hvs = [(wr2[...], br2[...]) for wr2, br2 in hs]
        owv, obv = owr[...], obr[...]
        if not packed:
            o = mlp_chain([xr[...] for xr in xs], wvs, b1v, hvs, owv, obv)
            out_refs[0][...] = o
            return
        xvals = [xr[...] for xr in xs]
        outs = []
        narrows = []
        for g in range(8):
            xvs = [xv[:, g * 16:(g + 1) * 16] for xv in xvals]
            o = mlp_chain(xvs, wvs, b1v, hvs, owv, obv)
            outs.append(o)
            if narrow_width:
                narrows.append(o[:, :narrow_width])
        out_refs[0][...] = jnp.concatenate(outs, axis=1)
        if narrow_width:
            out_refs[1][...] = jnp.concatenate(narrows, axis=1)

    if out_pack > 1:
        out_specs = [pl.BlockSpec((br // out_pack, 128), lambda i: (i, 0))]
        out_shape = [jax.ShapeDtypeStruct((n_out // out_pack, 128), F32)]
        if narrow_width:
            out_specs.append(pl.BlockSpec((br // 8, 8 * narrow_width),
                                          lambda i: (i, 0)))
            out_shape.append(
                jax.ShapeDtypeStruct((n_out // 8, 8 * narrow_width), F32))
    else:
        out_specs = [pl.BlockSpec((br, out_width), lambda i: (i, 0))]
        out_shape = [jax.ShapeDtypeStruct((n_out, out_width), F32)]

    res = pl.pallas_call(
        body,
        grid=(grid,),
        in_specs=in_specs,
        out_specs=out_specs,
        out_shape=out_shape,
    )(*operands)
    return res if (out_pack > 1 and narrow_width) else res[0]


def _pad_rows(wt, rows):
    return jnp.pad(wt, ((0, rows - wt.shape[0]), (0, 0)))


def _pad_cols(wt, cols):
    return jnp.pad(wt, ((0, 0), (0, cols - wt.shape[1])))


def kernel(x, edge_index, edge_attr, params):
    n_nodes = x.shape[0]
    n_edges = edge_index.shape[1]
    n_pad = ((n_nodes + NS * 8 - 1) // (NS * 8)) * (NS * 8)

    dst = edge_index[1].astype(jnp.int32)
    src = edge_index[0].astype(jnp.int32)
    idx_cat = jnp.concatenate([dst, src])

    xp = jnp.zeros((n_pad, DN), F32).at[:n_nodes, :3].set(x)
    eap = jnp.pad(edge_attr.astype(F32),
                  ((0, 0), (0, DN - 4))).reshape(n_edges // 8, 128)

    BR_E = 6400
    BR_N = n_pad // 16
    EBLK = n_edges // BR_E

    def bf(w):
        return w.astype(BF16)

    def rel_call(p, e_pieces, gp):
        (W1, b1), (W2, b2), (W3, b3), (W4, b4) = p
        W1t = W1.T
        pieces = [(gp, 0, bf(_pad_rows(W1t[0:3], DN)), 8),
                  (gp, EBLK, bf(_pad_rows(W1t[3:6], DN)), 8)]
        col = 6
        for arr, d, pack in e_pieces:
            wp = W1t[col:col + d]
            col += d
            lw = 128 // pack
            if lw != d:
                wp = _pad_rows(wp, lw)
            pieces.append((arr, 0, bf(wp), pack))
        hidden = [(bf(W2.T), b2[None, :]), (bf(W3.T), b3[None, :])]
        return _tc_mlp(n_edges, BR_E, pieces, b1[None, :], hidden,
                       bf(_pad_cols(W4.T, DN)), _pad_cols(b4[None, :], DN),
                       out_pack=8)

    def obj_call(p, xtab, agg):
        (W1, b1), (W2, b2), (W3, b3), (W4, b4) = p
        W1t = W1.T
        eo = W1t.shape[0] - 3
        wx = bf(_pad_rows(W1t[0:3], DN))
        wa = bf(_pad_rows(W1t[3:3 + eo], DN))
        pieces = [(xtab, 0, wx, 1), (agg, 0, wa, 1),
                  (agg, n_pad // BR_N, wa, 1)]
        hidden = [(bf(W2.T), b2[None, :]), (bf(W3.T), b3[None, :])]
        return _tc_mlp(n_pad, BR_N, pieces, b1[None, :], hidden,
                       bf(_pad_cols(W4.T, DN)), _pad_cols(b4[None, :], DN))

    def head_call(p, pieces_in, n_out, br, final, **kw):
        (W1, b1), (W2, b2), (W3, b3), (W4, b4) = p
        W1t = W1.T
        pieces = []
        col = 0
        for arr, d, pack in pieces_in:
            wp = W1t[col:col + d]
            col += d
            lw = 128 // pack if pack > 1 else arr.shape[1]
            if lw != d:
                wp = _pad_rows(wp, lw)
            pieces.append((arr, 0, bf(wp), pack))
        hidden = [(bf(W2.T), b2[None, :]), (bf(W3.T), b3[None, :])]
        wout, bout = W4.T, b4[None, :]
        if kw.get("out_pack", 1) > 1:
            wout, bout = _pad_cols(wout, DN), _pad_cols(bout, DN)
        return _tc_mlp(n_out, br, pieces, b1[None, :], hidden,
                       bf(wout), bout, final=final, **kw)

    g0 = _gather_rows(xp, idx_cat)
    gp0 = g0.reshape(2 * n_edges // 8, 128)
    e1 = rel_call(params['in_w1']['rel'], [(eap, 4, 8)], gp0)
    a1 = _segment_sum(e1.reshape(n_edges, DN), dst, n_pad)
    x1 = obj_call(params['in_w1']['obj'], xp, a1)

    g1 = _gather_rows(x1, idx_cat)
    gp1 = g1.reshape(2 * n_edges // 8, 128)
    e2 = rel_call(params['in_w2']['rel'], [(e1, 4, 8)], gp1)

    ew16, ewn = head_call(params['W'], [(eap, 4, 8), (e1, 4, 8), (e2, 4, 8)],
                         n_edges, BR_E, "sigmoid", out_pack=8,
                         narrow_width=1)

    ec1 = rel_call(params['in_c1']['rel'],
                   [(ew16, 1, 8), (eap, 4, 8), (e1, 4, 8), (e2, 4, 8)],
                   gp0)
    ac1 = _segment_sum(ec1.reshape(n_edges, DN), dst, n_pad)
    xc1 = obj_call(params['in_c1']['obj'], xp, ac1)

    gc2 = _gather_rows(xc1, idx_cat)
    gpc2 = gc2.reshape(2 * n_edges // 8, 128)
    ec2 = rel_call(params['in_c2']['rel'], [(ec1, 8, 8)], gpc2)
    ac2 = _segment_sum(ec2.reshape(n_edges, DN), dst, n_pad)
    xc2 = obj_call(params['in_c2']['obj'], xc1, ac2)

    gc3 = _gather_rows(xc2, idx_cat)
    gpc3 = gc3.reshape(2 * n_edges // 8, 128)
    ec3 = rel_call(params['in_c3']['rel'], [(ec2, 8, 8)], gpc3)
    ac3 = _segment_sum(ec3.reshape(n_edges, DN), dst, n_pad)
    xc3 = obj_call(params['in_c3']['obj'], xc2, ac3)

    node_pieces = [(xp, 3, 1), (xc1, 3, 1), (xc2, 3, 1), (xc3, 3, 1)]
    beta = head_call(params['B'], node_pieces, n_pad, BR_N, "sigmoid")
    xc = head_call(params['X'], node_pieces, n_pad, BR_N, None)

    ew = ewn.reshape(n_edges, 1)
    return ew, xc[:n_nodes], beta[:n_nodes]


# speedup vs baseline: 5.4337x; 1.4069x over previous
import functools

import jax
import jax.numpy as jnp
from jax import lax
from jax.experimental import pallas as pl
from jax.experimental.pallas import tpu as pltpu
from jax.experimental.pallas import tpu_sc as plsc

F32 = jnp.float32
BF16 = jnp.bfloat16

NC, NS = 2, 16
NW = NC * NS
DN = 16
G = 8


def _sc_mesh():
    return plsc.VectorSubcoreMesh(core_axis_name="c", subcore_axis_name="s",
                                  num_cores=NC, num_subcores=NS)


def _gather_rows(table, idx, chunk=2000):
    M = idx.shape[0]
    b_w = M // NW
    n_chunks = b_w // chunk

    @functools.partial(
        pl.kernel,
        out_type=jax.ShapeDtypeStruct((M, DN), F32),
        mesh=_sc_mesh(),
        scratch_types=[pltpu.VMEM((chunk,), jnp.int32),
                       pltpu.VMEM((chunk, DN), F32),
                       pltpu.SemaphoreType.DMA],
        compiler_params=pltpu.CompilerParams(use_tc_tiling_on_sc=False),
    )
    def k(table_hbm, idx_hbm, out_hbm, idx_v, rows_v, sem):
        wid = lax.axis_index("c") * NS + lax.axis_index("s")
        base = wid * b_w

        def body(i, carry):
            off = base + i * chunk
            pltpu.sync_copy(idx_hbm.at[pl.ds(off, chunk)], idx_v)
            pltpu.async_copy(table_hbm.at[idx_v], rows_v, sem).wait()
            pltpu.sync_copy(rows_v, out_hbm.at[pl.ds(off, chunk)])
            return carry

        lax.fori_loop(0, n_chunks, body, 0)

    return k(table, idx)


def _segment_sum(e, dst, n_pad, chunk=1000):
    M = e.shape[0]
    b_w = M // NW
    n_chunks = b_w // chunk
    rpt = n_pad // NS
    zeros = jnp.zeros((rpt, DN), F32)

    @functools.partial(
        pl.kernel,
        out_type=jax.ShapeDtypeStruct((NC * n_pad, DN), F32),
        mesh=_sc_mesh(),
        scratch_types=[pltpu.VMEM((chunk,), jnp.int32),
                       pltpu.VMEM((chunk, DN), F32),
                       pltpu.VMEM_SHARED((n_pad, DN), F32),
                       pltpu.SemaphoreType.DMA],
        compiler_params=pltpu.CompilerParams(use_tc_tiling_on_sc=False),
    )
    def k(e_hbm, dst_hbm, z_hbm, out_hbm, idx_v, rows_v, acc_sh, sem):
        c = lax.axis_index("c")
        s = lax.axis_index("s")
        pltpu.sync_copy(z_hbm, acc_sh.at[pl.ds(s * rpt, rpt)])
        plsc.subcore_barrier()
        base = (c * NS + s) * b_w

        def body(i, carry):
            off = base + i * chunk
            pltpu.sync_copy(dst_hbm.at[pl.ds(off, chunk)], idx_v)
            pltpu.sync_copy(e_hbm.at[pl.ds(off, chunk)], rows_v)
            pltpu.sync_copy(rows_v, acc_sh.at[idx_v], add=True)
            return carry

        lax.fori_loop(0, n_chunks, body, 0)
        plsc.subcore_barrier()
        pltpu.sync_copy(acc_sh.at[pl.ds(s * rpt, rpt)],
                        out_hbm.at[pl.ds(c * n_pad + s * rpt, rpt)])

    return k(e, dst, zeros)


BRP = 800


def _edge_mlp(mp, pieces, b1, hidden, out_wt, out_b, final=None,
              narrow=False):
    grid = mp // BRP
    in_specs = []
    operands = []
    for arr, off, _ in pieces:
        in_specs.append(pl.BlockSpec((BRP, 128), lambda i, o=off: (i + o, 0)))
        operands.append(arr)

    def whole(a):
        in_specs.append(pl.BlockSpec(a.shape, lambda i: (0, 0)))
        operands.append(a)

    for _, _, w in pieces:
        whole(w)
    whole(b1)
    for w, b in hidden:
        whole(w)
        whole(b)
    whole(out_wt)
    whole(out_b)

    n_p = len(pieces)
    n_h = len(hidden)

    def body(*refs):
        xs = refs[:n_p]
        wps = refs[n_p:2 * n_p]
        k = 2 * n_p
        b1r = refs[k]
        k += 1
        hs = []
        for _ in range(n_h):
            hs.append((refs[k], refs[k + 1]))
            k += 2
        owr, obr = refs[k], refs[k + 1]
        out_refs = refs[k + 2:]
        acc = None
        for xr, wr in zip(xs, wps):
            t = jnp.dot(xr[...].astype(BF16), wr[...],
                        preferred_element_type=F32)
            acc = t if acc is None else acc + t
        h = jnp.maximum(acc + b1r[...], 0.0)
        for wr2, br2 in hs:
            h = jnp.maximum(
                jnp.dot(h.astype(BF16), wr2[...],
                        preferred_element_type=F32) + br2[...], 0.0)
        o = jnp.dot(h.astype(BF16), owr[...],
                    preferred_element_type=F32) + obr[...]
        if final == "sigmoid":
            o = jax.nn.sigmoid(o)
        out_refs[0][...] = o
        if narrow:
            out_refs[1][...] = jnp.concatenate(
                [o[:, DN * j:DN * j + 1] for j in range(G)], axis=1)

    out_specs = [pl.BlockSpec((BRP, 128), lambda i: (i, 0))]
    out_shape = [jax.ShapeDtypeStruct((mp, 128), F32)]
    if narrow:
        out_specs.append(pl.BlockSpec((BRP, G), lambda i: (i, 0)))
        out_shape.append(jax.ShapeDtypeStruct((mp, G), F32))

    res = pl.pallas_call(
        body,
        grid=(grid,),
        in_specs=in_specs,
        out_specs=out_specs,
        out_shape=out_shape,
    )(*operands)
    return res if narrow else res[0]


def _node_mlp(n_out, br, pieces, b1, hidden, out_wt, out_b, final=None):
    grid = n_out // br
    in_specs = []
    operands = []
    for arr, blk_off, _ in pieces:
        in_specs.append(pl.BlockSpec((br, arr.shape[1]),
                                     lambda i, o=blk_off: (i + o, 0)))
        operands.append(arr)

    def whole(a):
        in_specs.append(pl.BlockSpec(a.shape, lambda i: (0, 0)))
        operands.append(a)

    for _, _, wp in pieces:
        whole(wp)
    whole(b1)
    for w, b in hidden:
        whole(w)
        whole(b)
    whole(out_wt)
    whole(out_b)

    n_p = len(pieces)
    n_h = len(hidden)
    out_width = out_wt.shape[1]

    def body(*refs):
        xs = refs[:n_p]
        wps = refs[n_p:2 * n_p]
        k = 2 * n_p
        b1r = refs[k]
        k += 1
        hs = []
        for _ in range(n_h):
            hs.append((refs[k], refs[k + 1]))
            k += 2
        owr, obr = refs[k], refs[k + 1]
        out_ref = refs[k + 2]
        acc = None
        for xr, wr in zip(xs, wps):
            t = jnp.dot(xr[...].astype(BF16), wr[...],
                        preferred_element_type=F32)
            acc = t if acc is None else acc + t
        h = jnp.maximum(acc + b1r[...], 0.0)
        for wr2, br2 in hs:
            h = jnp.maximum(
                jnp.dot(h.astype(BF16), wr2[...],
                        preferred_element_type=F32) + br2[...], 0.0)
        o = jnp.dot(h.astype(BF16), owr[...],
                    preferred_element_type=F32) + obr[...]
        if final == "sigmoid":
            o = jax.nn.sigmoid(o)
        out_ref[...] = o

    return pl.pallas_call(
        body,
        grid=(grid,),
        in_specs=in_specs,
        out_specs=pl.BlockSpec((br, out_width), lambda i: (i, 0)),
        out_shape=jax.ShapeDtypeStruct((n_out, out_width), F32),
    )(*operands)


def _repack_ea(ea, n_edges):
    mp = n_edges // G
    grid = mp // BRP
    gblk = mp // BRP

    def body(*refs):
        out_ref = refs[G]
        z = jnp.zeros((BRP, DN - 4), F32)
        out_ref[...] = jnp.concatenate(
            [jnp.concatenate([refs[g][...], z], axis=1) for g in range(G)],
            axis=1)

    return pl.pallas_call(
        body,
        grid=(grid,),
        in_specs=[pl.BlockSpec((BRP, 4), lambda i, g=g: (i + g * gblk, 0))
                  for g in range(G)],
        out_specs=pl.BlockSpec((BRP, 128), lambda i: (i, 0)),
        out_shape=jax.ShapeDtypeStruct((mp, 128), F32),
    )(*([ea] * G))


def _pad_rows(wt, rows):
    return jnp.pad(wt, ((0, rows - wt.shape[0]), (0, 0)))


def _pad_cols(wt, cols):
    return jnp.pad(wt, ((0, 0), (0, cols - wt.shape[1])))


def _bd(w):
    return jnp.kron(jnp.eye(G, dtype=w.dtype), w)


def kernel(x, edge_index, edge_attr, params):
    n_nodes = x.shape[0]
    n_edges = edge_index.shape[1]
    n_pad = ((n_nodes + NS * 8 - 1) // (NS * 8)) * (NS * 8)
    me = n_edges // G
    mg = 2 * n_edges // G
    OB = (n_edges // G) // BRP

    dst = edge_index[1].astype(jnp.int32)
    src = edge_index[0].astype(jnp.int32)
    idx_cat = jnp.concatenate([dst, src])
    idx_p = jnp.transpose(idx_cat.reshape(G, mg)).reshape(2 * n_edges)
    dst_p = jnp.transpose(dst.reshape(G, n_edges // G)).reshape(n_edges)

    xp = jnp.zeros((n_pad, DN), F32).at[:n_nodes, :3].set(x)
    eap = _repack_ea(edge_attr.astype(F32), n_edges)

    BR_N = n_pad // 16

    def bf(w):
        return w.astype(BF16)

    def rel_first_gather_w(W1t, h):
        Wd = _pad_rows(W1t[0:3], DN)
        Ws = _pad_rows(W1t[3:6], DN)
        WA = jnp.zeros((128, G * h), F32)
        WB = jnp.zeros((128, G * h), F32)
        for a in range(4):
            WA = WA.at[DN * a:DN * a + DN, 2 * a * h:(2 * a + 1) * h].set(Wd)
            WA = WA.at[64 + DN * a:64 + DN * a + DN,
                       2 * a * h:(2 * a + 1) * h].set(Ws)
            WB = WB.at[DN * a:DN * a + DN,
                       (2 * a + 1) * h:(2 * a + 2) * h].set(Wd)
            WB = WB.at[64 + DN * a:64 + DN * a + DN,
                       (2 * a + 1) * h:(2 * a + 2) * h].set(Ws)
        return WA, WB

    def rel_call(p, e_pieces, gpk):
        (W1, b1), (W2, b2), (W3, b3), (W4, b4) = p
        h = W1.shape[0]
        W1t = W1.T
        WA, WB = rel_first_gather_w(W1t, h)
        pieces = [(gpk, 0, bf(WA)), (gpk, OB, bf(WB))]
        col = 6
        for arr, d in e_pieces:
            wp = _pad_rows(W1t[col:col + d], DN)
            col += d
            pieces.append((arr, 0, bf(_bd(wp))))
        hidden = [(bf(_bd(W2.T)), jnp.tile(b2[None, :], (1, G))),
                  (bf(_bd(W3.T)), jnp.tile(b3[None, :], (1, G)))]
        owt = bf(_bd(_pad_cols(W4.T, DN)))
        ob = jnp.tile(_pad_cols(b4[None, :], DN), (1, G))
        return _edge_mlp(me, pieces, jnp.tile(b1[None, :], (1, G)),
                         hidden, owt, ob)

    def whead_call(p, e_pieces):
        (W1, b1), (W2, b2), (W3, b3), (W4, b4) = p
        h = W1.shape[0]
        W1t = W1.T
        pieces = []
        col = 0
        for arr, d in e_pieces:
            wp = _pad_rows(W1t[col:col + d], DN)
            col += d
            pieces.append((arr, 0, bf(_bd(wp))))
        hidden = [(bf(_bd(W2.T)), jnp.tile(b2[None, :], (1, G))),
                  (bf(_bd(W3.T)), jnp.tile(b3[None, :], (1, G)))]
        owt = bf(_bd(_pad_cols(W4.T, DN)))
        ob = jnp.tile(_pad_cols(b4[None, :], DN), (1, G))
        return _edge_mlp(me, pieces, jnp.tile(b1[None, :], (1, G)),
                         hidden, owt, ob, final="sigmoid", narrow=True)

    def obj_call(p, xtab, agg):
        (W1, b1), (W2, b2), (W3, b3), (W4, b4) = p
        W1t = W1.T
        eo = W1t.shape[0] - 3
        wx = bf(_pad_rows(W1t[0:3], DN))
        wa = bf(_pad_rows(W1t[3:3 + eo], DN))
        pieces = [(xtab, 0, wx), (agg, 0, wa), (agg, n_pad // BR_N, wa)]
        hidden = [(bf(W2.T), b2[None, :]), (bf(W3.T), b3[None, :])]
        return _node_mlp(n_pad, BR_N, pieces, b1[None, :], hidden,
                         bf(_pad_cols(W4.T, DN)), _pad_cols(b4[None, :], DN))

    def nhead_call(p, tabs, final):
        (W1, b1), (W2, b2), (W3, b3), (W4, b4) = p
        W1t = W1.T
        pieces = [(tab, 0, bf(_pad_rows(W1t[3 * i:3 * i + 3], DN)))
                  for i, tab in enumerate(tabs)]
        hidden = [(bf(W2.T), b2[None, :]), (bf(W3.T), b3[None, :])]
        return _node_mlp(n_pad, BR_N, pieces, b1[None, :], hidden,
                         bf(W4.T), b4[None, :], final=final)

    g0 = _gather_rows(xp, idx_p)
    gp0 = g0.reshape(mg, 128)
    e1 = rel_call(params['in_w1']['rel'], [(eap, 4)], gp0)
    a1 = _segment_sum(e1.reshape(n_edges, DN), dst_p, n_pad)
    x1 = obj_call(params['in_w1']['obj'], xp, a1)

    g1 = _gather_rows(x1, idx_p)
    gp1 = g1.reshape(mg, 128)
    e2 = rel_call(params['in_w2']['rel'], [(e1, 4)], gp1)

    ew16, ewn = whead_call(params['W'], [(eap, 4), (e1, 4), (e2, 4)])

    ec1 = rel_call(params['in_c1']['rel'],
                   [(ew16, 1), (eap, 4), (e1, 4), (e2, 4)], gp0)
    ac1 = _segment_sum(ec1.reshape(n_edges, DN), dst_p, n_pad)
    xc1 = obj_call(params['in_c1']['obj'], xp, ac1)

    gc2 = _gather_rows(xc1, idx_p)
    gpc2 = gc2.reshape(mg, 128)
    ec2 = rel_call(params['in_c2']['rel'], [(ec1, 8)], gpc2)
    ac2 = _segment_sum(ec2.reshape(n_edges, DN), dst_p, n_pad)
    xc2 = obj_call(params['in_c2']['obj'], xc1, ac2)

    gc3 = _gather_rows(xc2, idx_p)
    gpc3 = gc3.reshape(mg, 128)
    ec3 = rel_call(params['in_c3']['rel'], [(ec2, 8)], gpc3)
    ac3 = _segment_sum(ec3.reshape(n_edges, DN), dst_p, n_pad)
    xc3 = obj_call(params['in_c3']['obj'], xc2, ac3)

    tabs = [xp, xc1, xc2, xc3]
    beta = nhead_call(params['B'], tabs, "sigmoid")
    xc = nhead_call(params['X'], tabs, None)

    ew = jnp.transpose(ewn).reshape(n_edges, 1)
    return ew, xc[:n_nodes], beta[:n_nodes]
